# baseline (speedup 1.0000x reference)
import functools
import math

import jax
import jax.numpy as jnp
import numpy as np
from jax import lax
from jax.experimental import pallas as pl
from jax.experimental.pallas import tpu as pltpu
from jax.experimental.pallas import tpu_sc as plsc

N_NODES = 10000
N_EDGES = 160000
HID = 256
HEADS = 8
DH = HID // HEADS
SCALE = math.sqrt(DH)

NC = 2
NS = 16
NW = NC * NS
NCHK = 5
CE = N_EDGES // NCHK
EW = CE // NW
CH = 128
NCH = EW // CH
CT = EW - NCH * CH

NP_NODES = 10240
EXW = 128
TN = 1000
TE = 1600


def _np_consts():
    d = np.arange(HID) // DH
    O = np.zeros((3, HID, 3 * HEADS), np.float32)
    for t in range(3):
        for c in range(HEADS):
            O[t, :, 8 * t + c] = (d == c)
    E1 = np.zeros((24, EXW), np.float32)
    E2 = np.zeros((24, EXW), np.float32)
    for j in range(24):
        E1[j, j] = 1.0
        E2[j, 24 + j] = 1.0
    selA = np.zeros((EXW, 24), np.float32)
    selB = np.zeros((EXW, 24), np.float32)
    for j in range(24):
        selA[j, j] = 1.0
        selB[24 + j, j] = 1.0
    sel2 = np.zeros((24, HID), np.float32)
    for j in range(24):
        sel2[j, :] = (d == (j % HEADS))
    return O, E1, E2, selA, selB, sel2


_O, _E1, _E2, _SELA, _SELB, _SEL2 = _np_consts()


def _qv_body(nh, wqt, bq, wvt, bv, q_out, v_out):
    x = nh[...]
    q_out[...] = jnp.dot(x, wqt[...], preferred_element_type=jnp.float32) + bq[...]
    v_out[...] = jnp.dot(x, wvt[...], preferred_element_type=jnp.float32) + bv[...]


def _qv(node_h, wqt, bq, wvt, bv):
    n = node_h.shape[0]
    grid = n // TN
    return pl.pallas_call(
        _qv_body,
        grid=(grid,),
        in_specs=[
            pl.BlockSpec((TN, HID), lambda i: (i, 0)),
            pl.BlockSpec((HID, HID), lambda i: (0, 0)),
            pl.BlockSpec((1, HID), lambda i: (0, 0)),
            pl.BlockSpec((HID, HID), lambda i: (0, 0)),
            pl.BlockSpec((1, HID), lambda i: (0, 0)),
        ],
        out_specs=[
            pl.BlockSpec((TN, HID), lambda i: (i, 0)),
            pl.BlockSpec((TN, HID), lambda i: (i, 0)),
        ],
        out_shape=[
            jax.ShapeDtypeStruct((n, HID), jnp.float32),
            jax.ShapeDtypeStruct((n, HID), jnp.float32),
        ],
    )(node_h, wqt, bq, wvt, bv)


def _gather_body(q_hbm, src_hbm, dst_hbm, qs_out, qd_out, idx_v, rows_v,
                 idx_t, rows_t, sem):
    c = lax.axis_index("c")
    s = lax.axis_index("s")
    wid = s * NC + c
    base = wid * EW

    def step(i, carry):
        off = base + i * CH
        pltpu.sync_copy(src_hbm.at[pl.ds(off, CH)], idx_v)
        pltpu.async_copy(q_hbm.at[idx_v], rows_v, sem).wait()
        pltpu.sync_copy(rows_v, qs_out.at[pl.ds(off, CH)])
        pltpu.sync_copy(dst_hbm.at[pl.ds(off, CH)], idx_v)
        pltpu.async_copy(q_hbm.at[idx_v], rows_v, sem).wait()
        pltpu.sync_copy(rows_v, qd_out.at[pl.ds(off, CH)])
        return carry

    lax.fori_loop(0, NCH, step, 0)
    off = base + NCH * CH
    pltpu.sync_copy(src_hbm.at[pl.ds(off, CT)], idx_t)
    pltpu.async_copy(q_hbm.at[idx_t], rows_t, sem).wait()
    pltpu.sync_copy(rows_t, qs_out.at[pl.ds(off, CT)])
    pltpu.sync_copy(dst_hbm.at[pl.ds(off, CT)], idx_t)
    pltpu.async_copy(q_hbm.at[idx_t], rows_t, sem).wait()
    pltpu.sync_copy(rows_t, qd_out.at[pl.ds(off, CT)])


def _gather(q, src, dst):
    mesh = plsc.VectorSubcoreMesh(core_axis_name="c", subcore_axis_name="s")
    f = functools.partial(
        pl.kernel,
        out_type=(
            jax.ShapeDtypeStruct((CE, HID), jnp.float32),
            jax.ShapeDtypeStruct((CE, HID), jnp.float32),
        ),
        mesh=mesh,
        scratch_types=[
            pltpu.VMEM((CH,), jnp.int32),
            pltpu.VMEM((CH, HID), jnp.float32),
            pltpu.VMEM((CT,), jnp.int32),
            pltpu.VMEM((CT, HID), jnp.float32),
            pltpu.SemaphoreType.DMA,
        ],
    )(_gather_body)
    return f(q, src, dst)


def _dots_body(eh, qs, qd, dist, wkt, bk, o_in, o_out, o_diag, e1, e2, lam, ex_out):
    k = jnp.dot(eh[...], wkt[...], preferred_element_type=jnp.float32) + bk[...]
    a = qs[...] * k
    b = qd[...] * k
    cdg = qs[...] * qd[...]
    m24 = (
        jnp.dot(a, o_in[...], preferred_element_type=jnp.float32)
        + jnp.dot(b, o_out[...], preferred_element_type=jnp.float32)
        + jnp.dot(cdg, o_diag[...], preferred_element_type=jnp.float32)
    )
    e24 = jnp.exp(m24 * (1.0 / SCALE))
    lamv = jnp.clip(lam[...], 0.0, 1.0)
    df = jnp.exp(lamv * jnp.log(dist[...]))
    ws24 = e24 * df
    ex_out[...] = (
        jnp.dot(e24, e1[...], preferred_element_type=jnp.float32)
        + jnp.dot(ws24, e2[...], preferred_element_type=jnp.float32)
    )


def _dots(edge_h, qs, qd, dist, wkt, bk, lam):
    grid = CE // TE
    return pl.pallas_call(
        _dots_body,
        grid=(grid,),
        in_specs=[
            pl.BlockSpec((TE, HID), lambda i: (i, 0)),
            pl.BlockSpec((TE, HID), lambda i: (i, 0)),
            pl.BlockSpec((TE, HID), lambda i: (i, 0)),
            pl.BlockSpec((TE, 1), lambda i: (i, 0)),
            pl.BlockSpec((HID, HID), lambda i: (0, 0)),
            pl.BlockSpec((1, HID), lambda i: (0, 0)),
            pl.BlockSpec((HID, 24), lambda i: (0, 0)),
            pl.BlockSpec((HID, 24), lambda i: (0, 0)),
            pl.BlockSpec((HID, 24), lambda i: (0, 0)),
            pl.BlockSpec((24, EXW), lambda i: (0, 0)),
            pl.BlockSpec((24, EXW), lambda i: (0, 0)),
            pl.BlockSpec((1, 1), lambda i: (0, 0)),
        ],
        out_specs=pl.BlockSpec((TE, EXW), lambda i: (i, 0)),
        out_shape=jax.ShapeDtypeStruct((CE, EXW), jnp.float32),
    )(
        edge_h, qs, qd, dist, wkt, bk,
        jnp.asarray(_O[0]), jnp.asarray(_O[1]), jnp.asarray(_O[2]),
        jnp.asarray(_E1), jnp.asarray(_E2), lam,
    )


def _scatter_body(ex0, ex1, ex2, ex3, ex4, dst_hbm, zeros_hbm, p0_out, p1_out,
                  accum, idx_v, ex_v, idx_t, ex_t):
    c = lax.axis_index("c")
    s = lax.axis_index("s")
    wid = s * NC + c

    @pl.when(s == 0)
    def _():
        pltpu.sync_copy(zeros_hbm, accum)

    plsc.subcore_barrier()

    for j, ex_hbm in enumerate((ex0, ex1, ex2, ex3, ex4)):
        def step(i, carry, ex_hbm=ex_hbm, j=j):
            off = wid * EW + i * CH
            pltpu.sync_copy(dst_hbm.at[pl.ds(j * CE + off, CH)], idx_v)
            pltpu.sync_copy(ex_hbm.at[pl.ds(off, CH)], ex_v)
            pltpu.sync_copy(ex_v, accum.at[idx_v], add=True)
            return carry

        lax.fori_loop(0, NCH, step, 0)
        off = wid * EW + NCH * CH
        pltpu.sync_copy(dst_hbm.at[pl.ds(j * CE + off, CT)], idx_t)
        pltpu.sync_copy(ex_hbm.at[pl.ds(off, CT)], ex_t)
        pltpu.sync_copy(ex_t, accum.at[idx_t], add=True)
    plsc.subcore_barrier()

    rows = NP_NODES // NS

    @pl.when(c == 0)
    def _():
        pltpu.sync_copy(accum.at[pl.ds(s * rows, rows)], p0_out.at[pl.ds(s * rows, rows)])

    @pl.when(c == 1)
    def _():
        pltpu.sync_copy(accum.at[pl.ds(s * rows, rows)], p1_out.at[pl.ds(s * rows, rows)])


def _scatter(exs, dst, zeros):
    mesh = plsc.VectorSubcoreMesh(core_axis_name="c", subcore_axis_name="s")
    f = functools.partial(
        pl.kernel,
        out_type=(
            jax.ShapeDtypeStruct((NP_NODES, EXW), jnp.float32),
            jax.ShapeDtypeStruct((NP_NODES, EXW), jnp.float32),
        ),
        mesh=mesh,
        scratch_types=[
            pltpu.VMEM_SHARED((NP_NODES, EXW), jnp.float32),
            pltpu.VMEM((CH,), jnp.int32),
            pltpu.VMEM((CH, EXW), jnp.float32),
            pltpu.VMEM((CT,), jnp.int32),
            pltpu.VMEM((CT, EXW), jnp.float32),
        ],
    )(_scatter_body)
    return f(*exs, dst, zeros)


def _final_body(p0, p1, v, sela, selb, sel2, wot, bo, out):
    p = p0[...] + p1[...]
    den = jnp.dot(p, sela[...], preferred_element_type=jnp.float32)
    ws = jnp.dot(p, selb[...], preferred_element_type=jnp.float32)
    ratio = jnp.where(den > 0.0, ws / den, 0.0)
    sexp = jnp.dot(ratio, sel2[...], preferred_element_type=jnp.float32)
    h = v[...] * sexp
    o = jnp.dot(h, wot[...], preferred_element_type=jnp.float32) + bo[...]
    out[...] = jnp.where(o >= 0.0, o, 0.1 * o)


def _final(p0, p1, v, wot, bo):
    grid = N_NODES // TN
    return pl.pallas_call(
        _final_body,
        grid=(grid,),
        in_specs=[
            pl.BlockSpec((TN, EXW), lambda i: (i, 0)),
            pl.BlockSpec((TN, EXW), lambda i: (i, 0)),
            pl.BlockSpec((TN, HID), lambda i: (i, 0)),
            pl.BlockSpec((EXW, 24), lambda i: (0, 0)),
            pl.BlockSpec((EXW, 24), lambda i: (0, 0)),
            pl.BlockSpec((24, HID), lambda i: (0, 0)),
            pl.BlockSpec((HID, HID), lambda i: (0, 0)),
            pl.BlockSpec((1, HID), lambda i: (0, 0)),
        ],
        out_specs=pl.BlockSpec((TN, HID), lambda i: (i, 0)),
        out_shape=jax.ShapeDtypeStruct((N_NODES, HID), jnp.float32),
    )(p0, p1, v, jnp.asarray(_SELA), jnp.asarray(_SELB), jnp.asarray(_SEL2), wot, bo)


def kernel(node_h, edge_h, edge_distance, edge_index, W_q, b_q, W_k, b_k,
           W_v, b_v, W_o, b_o, attenuation_lambda):
    src = edge_index[0].astype(jnp.int32)
    dst = edge_index[1].astype(jnp.int32)
    q, v = _qv(node_h, W_q.T, b_q.reshape(1, -1), W_v.T, b_v.reshape(1, -1))
    wkt = W_k.T
    bk = b_k.reshape(1, -1)
    lam = attenuation_lambda.reshape(1, 1)
    exs = []
    for j in range(NCHK):
        sl = slice(j * CE, (j + 1) * CE)
        qs_j, qd_j = _gather(q, src[sl], dst[sl])
        exs.append(_dots(edge_h[sl], qs_j, qd_j, edge_distance[sl], wkt, bk, lam))
    zeros = jnp.zeros((NP_NODES, EXW), jnp.float32)
    p0, p1 = _scatter(exs, dst, zeros)
    return _final(p0, p1, v, W_o.T, b_o.reshape(1, -1))


# speedup vs baseline: 1.1575x; 1.1575x over previous
import functools
import math

import jax
import jax.numpy as jnp
import numpy as np
from jax import lax
from jax.experimental import pallas as pl
from jax.experimental.pallas import tpu as pltpu
from jax.experimental.pallas import tpu_sc as plsc

N_NODES = 10000
N_EDGES = 160000
HID = 256
HEADS = 8
DH = HID // HEADS
SCALE = math.sqrt(DH)

NC = 2
NS = 16
NW = NC * NS
NCHK = 5
CE = N_EDGES // NCHK
EW = CE // NW
CH = 128
NCH = EW // CH
CT = EW - NCH * CH

NP_NODES = 10240
EXW = 128
EXC = EXW
TN = 1000
TE = 1600


def _np_consts():
    d = np.arange(HID) // DH
    O = np.zeros((3, HID, 3 * HEADS), np.float32)
    for t in range(3):
        for c in range(HEADS):
            O[t, :, 8 * t + c] = (d == c)
    E1 = np.zeros((24, EXC), np.float32)
    E2 = np.zeros((24, EXC), np.float32)
    for j in range(24):
        E1[j, j] = 1.0
        E2[j, 24 + j] = 1.0
    selA = np.zeros((EXW, 24), np.float32)
    selB = np.zeros((EXW, 24), np.float32)
    for j in range(24):
        selA[j, j] = 1.0
        selB[24 + j, j] = 1.0
    sel2 = np.zeros((24, HID), np.float32)
    for j in range(24):
        sel2[j, :] = (d == (j % HEADS))
    return O, E1, E2, selA, selB, sel2


_O, _E1, _E2, _SELA, _SELB, _SEL2 = _np_consts()


def _qv_body(nh, wqt, bq, wvt, bv, q_out, v_out):
    x = nh[...]
    q_out[...] = jnp.dot(x, wqt[...], preferred_element_type=jnp.float32) + bq[...]
    v_out[...] = jnp.dot(x, wvt[...], preferred_element_type=jnp.float32) + bv[...]


def _qv(node_h, wqt, bq, wvt, bv):
    n = node_h.shape[0]
    grid = n // TN
    return pl.pallas_call(
        _qv_body,
        grid=(grid,),
        in_specs=[
            pl.BlockSpec((TN, HID), lambda i: (i, 0)),
            pl.BlockSpec((HID, HID), lambda i: (0, 0)),
            pl.BlockSpec((1, HID), lambda i: (0, 0)),
            pl.BlockSpec((HID, HID), lambda i: (0, 0)),
            pl.BlockSpec((1, HID), lambda i: (0, 0)),
        ],
        out_specs=[
            pl.BlockSpec((TN, HID), lambda i: (i, 0)),
            pl.BlockSpec((TN, HID), lambda i: (i, 0)),
        ],
        out_shape=[
            jax.ShapeDtypeStruct((n, HID), jnp.float32),
            jax.ShapeDtypeStruct((n, HID), jnp.float32),
        ],
    )(node_h, wqt, bq, wvt, bv)


def _make_gather(j):
    def body(q_hbm, src_hbm, dst_hbm, qs_out, qd_out,
             idx_a, idx_b, rows_a, rows_b, idx_t, rows_t, sem_a, sem_b):
        c = lax.axis_index("c")
        s = lax.axis_index("s")
        wid = s * NC + c
        gbase = j * CE + wid * EW
        obase = wid * EW

        jobs = []
        for r in range(NCH):
            jobs.append((src_hbm, qs_out, r))
            jobs.append((dst_hbm, qd_out, r))
        idxs = (idx_a, idx_b)
        rows = (rows_a, rows_b)
        sems = (sem_a, sem_b)

        descs = [None] * len(jobs)

        def start(t):
            tab, _, r = jobs[t]
            b = t % 2
            pltpu.sync_copy(tab.at[pl.ds(gbase + r * CH, CH)], idxs[b])
            descs[t] = pltpu.async_copy(q_hbm.at[idxs[b]], rows[b], sems[b])

        start(0)
        start(1)
        for t in range(len(jobs)):
            _, out, r = jobs[t]
            b = t % 2
            descs[t].wait()
            pltpu.sync_copy(rows[b], out.at[pl.ds(obase + r * CH, CH)])
            if t + 2 < len(jobs):
                start(t + 2)

        off = NCH * CH
        pltpu.sync_copy(src_hbm.at[pl.ds(gbase + off, CT)], idx_t)
        pltpu.async_copy(q_hbm.at[idx_t], rows_t, sem_a).wait()
        pltpu.sync_copy(rows_t, qs_out.at[pl.ds(obase + off, CT)])
        pltpu.sync_copy(dst_hbm.at[pl.ds(gbase + off, CT)], idx_t)
        pltpu.async_copy(q_hbm.at[idx_t], rows_t, sem_a).wait()
        pltpu.sync_copy(rows_t, qd_out.at[pl.ds(obase + off, CT)])

    return body


def _gather(j, q, src, dst):
    mesh = plsc.VectorSubcoreMesh(core_axis_name="c", subcore_axis_name="s")
    f = functools.partial(
        pl.kernel,
        out_type=(
            jax.ShapeDtypeStruct((CE, HID), jnp.float32),
            jax.ShapeDtypeStruct((CE, HID), jnp.float32),
        ),
        mesh=mesh,
        scratch_types=[
            pltpu.VMEM((CH,), jnp.int32),
            pltpu.VMEM((CH,), jnp.int32),
            pltpu.VMEM((CH, HID), jnp.float32),
            pltpu.VMEM((CH, HID), jnp.float32),
            pltpu.VMEM((CT,), jnp.int32),
            pltpu.VMEM((CT, HID), jnp.float32),
            pltpu.SemaphoreType.DMA,
            pltpu.SemaphoreType.DMA,
        ],
    )(_make_gather(j))
    return f(q, src, dst)


def _dots_body(eh, qs, qd, dist, wkt, bk, o_in, o_out, o_diag, e1, e2, lam, ex_out):
    k = jnp.dot(eh[...], wkt[...], preferred_element_type=jnp.float32) + bk[...]
    a = qs[...] * k
    b = qd[...] * k
    cdg = qs[...] * qd[...]
    m24 = (
        jnp.dot(a, o_in[...], preferred_element_type=jnp.float32)
        + jnp.dot(b, o_out[...], preferred_element_type=jnp.float32)
        + jnp.dot(cdg, o_diag[...], preferred_element_type=jnp.float32)
    )
    e24 = jnp.exp(m24 * (1.0 / SCALE))
    lamv = jnp.clip(lam[...], 0.0, 1.0)
    df = jnp.exp(lamv * jnp.log(dist[...]))
    ws24 = e24 * df
    ex_out[...] = (
        jnp.dot(e24, e1[...], preferred_element_type=jnp.float32)
        + jnp.dot(ws24, e2[...], preferred_element_type=jnp.float32)
    )


def _dots(j, edge_h, qs, qd, dist, wkt, bk, lam):
    grid = CE // TE
    boff = j * (CE // TE)
    return pl.pallas_call(
        _dots_body,
        grid=(grid,),
        in_specs=[
            pl.BlockSpec((TE, HID), lambda i: (boff + i, 0)),
            pl.BlockSpec((TE, HID), lambda i: (i, 0)),
            pl.BlockSpec((TE, HID), lambda i: (i, 0)),
            pl.BlockSpec((TE, 1), lambda i: (boff + i, 0)),
            pl.BlockSpec((HID, HID), lambda i: (0, 0)),
            pl.BlockSpec((1, HID), lambda i: (0, 0)),
            pl.BlockSpec((HID, 24), lambda i: (0, 0)),
            pl.BlockSpec((HID, 24), lambda i: (0, 0)),
            pl.BlockSpec((HID, 24), lambda i: (0, 0)),
            pl.BlockSpec((24, EXC), lambda i: (0, 0)),
            pl.BlockSpec((24, EXC), lambda i: (0, 0)),
            pl.BlockSpec((1, 1), lambda i: (0, 0)),
        ],
        out_specs=pl.BlockSpec((TE, EXC), lambda i: (i, 0)),
        out_shape=jax.ShapeDtypeStruct((CE, EXC), jnp.float32),
    )(
        edge_h, qs, qd, dist, wkt, bk,
        jnp.asarray(_O[0]), jnp.asarray(_O[1]), jnp.asarray(_O[2]),
        jnp.asarray(_E1), jnp.asarray(_E2), lam,
    )


def _scatter_body(ex0, ex1, ex2, ex3, ex4, dst_hbm, zeros_hbm, p0_out, p1_out,
                  accum, idx_v, ex_v, idx_t, ex_t):
    c = lax.axis_index("c")
    s = lax.axis_index("s")
    wid = s * NC + c

    @pl.when(s == 0)
    def _():
        pltpu.sync_copy(zeros_hbm, accum)

    plsc.subcore_barrier()

    for j, ex_hbm in enumerate((ex0, ex1, ex2, ex3, ex4)):
        def step(i, carry, ex_hbm=ex_hbm, j=j):
            off = wid * EW + i * CH
            pltpu.sync_copy(dst_hbm.at[pl.ds(j * CE + off, CH)], idx_v)
            pltpu.sync_copy(ex_hbm.at[pl.ds(off, CH)], ex_v)
            pltpu.sync_copy(ex_v, accum.at[idx_v], add=True)
            return carry

        lax.fori_loop(0, NCH, step, 0)
        off = wid * EW + NCH * CH
        pltpu.sync_copy(dst_hbm.at[pl.ds(j * CE + off, CT)], idx_t)
        pltpu.sync_copy(ex_hbm.at[pl.ds(off, CT)], ex_t)
        pltpu.sync_copy(ex_t, accum.at[idx_t], add=True)
    plsc.subcore_barrier()

    rows = NP_NODES // NS

    @pl.when(c == 0)
    def _():
        pltpu.sync_copy(accum.at[pl.ds(s * rows, rows)], p0_out.at[pl.ds(s * rows, rows)])

    @pl.when(c == 1)
    def _():
        pltpu.sync_copy(accum.at[pl.ds(s * rows, rows)], p1_out.at[pl.ds(s * rows, rows)])


def _scatter(exs, dst, zeros):
    mesh = plsc.VectorSubcoreMesh(core_axis_name="c", subcore_axis_name="s")
    f = functools.partial(
        pl.kernel,
        out_type=(
            jax.ShapeDtypeStruct((NP_NODES, EXW), jnp.float32),
            jax.ShapeDtypeStruct((NP_NODES, EXW), jnp.float32),
        ),
        mesh=mesh,
        scratch_types=[
            pltpu.VMEM_SHARED((NP_NODES, EXW), jnp.float32),
            pltpu.VMEM((CH,), jnp.int32),
            pltpu.VMEM((CH, EXW), jnp.float32),
            pltpu.VMEM((CT,), jnp.int32),
            pltpu.VMEM((CT, EXW), jnp.float32),
        ],
    )(_scatter_body)
    return f(*exs, dst, zeros)


def _final_body(p0, p1, v, sela, selb, sel2, wot, bo, out):
    p = p0[...] + p1[...]
    den = jnp.dot(p, sela[...], preferred_element_type=jnp.float32)
    ws = jnp.dot(p, selb[...], preferred_element_type=jnp.float32)
    ratio = jnp.where(den > 0.0, ws / den, 0.0)
    sexp = jnp.dot(ratio, sel2[...], preferred_element_type=jnp.float32)
    h = v[...] * sexp
    o = jnp.dot(h, wot[...], preferred_element_type=jnp.float32) + bo[...]
    out[...] = jnp.where(o >= 0.0, o, 0.1 * o)


def _final(p0, p1, v, wot, bo):
    grid = N_NODES // TN
    return pl.pallas_call(
        _final_body,
        grid=(grid,),
        in_specs=[
            pl.BlockSpec((TN, EXW), lambda i: (i, 0)),
            pl.BlockSpec((TN, EXW), lambda i: (i, 0)),
            pl.BlockSpec((TN, HID), lambda i: (i, 0)),
            pl.BlockSpec((EXW, 24), lambda i: (0, 0)),
            pl.BlockSpec((EXW, 24), lambda i: (0, 0)),
            pl.BlockSpec((24, HID), lambda i: (0, 0)),
            pl.BlockSpec((HID, HID), lambda i: (0, 0)),
            pl.BlockSpec((1, HID), lambda i: (0, 0)),
        ],
        out_specs=pl.BlockSpec((TN, HID), lambda i: (i, 0)),
        out_shape=jax.ShapeDtypeStruct((N_NODES, HID), jnp.float32),
    )(p0, p1, v, jnp.asarray(_SELA), jnp.asarray(_SELB), jnp.asarray(_SEL2), wot, bo)


def kernel(node_h, edge_h, edge_distance, edge_index, W_q, b_q, W_k, b_k,
           W_v, b_v, W_o, b_o, attenuation_lambda):
    src = edge_index[0].astype(jnp.int32)
    dst = edge_index[1].astype(jnp.int32)
    q, v = _qv(node_h, W_q.T, b_q.reshape(1, -1), W_v.T, b_v.reshape(1, -1))
    wkt = W_k.T
    bk = b_k.reshape(1, -1)
    lam = attenuation_lambda.reshape(1, 1)
    exs = []
    for j in range(NCHK):
        qs_j, qd_j = _gather(j, q, src, dst)
        exs.append(_dots(j, edge_h, qs_j, qd_j, edge_distance, wkt, bk, lam))
    zeros = jnp.zeros((NP_NODES, EXW), jnp.float32)
    p0, p1 = _scatter(exs, dst, zeros)
    return _final(p0, p1, v, W_o.T, b_o.reshape(1, -1))
